# Initial kernel scaffold; baseline (speedup 1.0000x reference)
import jax
import jax.numpy as jnp
from jax.experimental import pallas as pl


def kernel(input_data, sa1_params, sa2_params, sa3_params):
    raise NotImplementedError("write your pallas kernel here")


# speedup vs baseline: 3.3104x; 3.3104x over previous
import functools

import jax
import jax.numpy as jnp
from jax.experimental import pallas as pl
from jax.experimental.pallas import tpu as pltpu

_EPS = 1e-5


def _fps_body(S, N, xyzT_ref, out_ref):
    xr = xyzT_ref[0, 0:1, :]
    yr = xyzT_ref[0, 1:2, :]
    zr = xyzT_ref[0, 2:3, :]
    lane = jax.lax.broadcasted_iota(jnp.int32, (1, N), 1)

    def body(i, carry):
        dist, f = carry
        oh = lane == f
        cx = jnp.sum(jnp.where(oh, xr, 0.0))
        cy = jnp.sum(jnp.where(oh, yr, 0.0))
        cz = jnp.sum(jnp.where(oh, zr, 0.0))
        row = jnp.concatenate(
            [cx.reshape(1, 1), cy.reshape(1, 1), cz.reshape(1, 1)], axis=1)
        out_ref[0, pl.ds(i, 1), :] = row
        d = (xr - cx) ** 2
        d = d + (yr - cy) ** 2
        d = d + (zr - cz) ** 2
        dist = jnp.minimum(dist, d)
        m = jnp.max(dist)
        f2 = jnp.min(jnp.where(dist == m, lane, N))
        return dist, f2

    dist0 = jnp.full((1, N), 1e10, dtype=jnp.float32)
    jax.lax.fori_loop(0, S, body, (dist0, 0))


def _fps(xyzT, S):
    B, _, N = xyzT.shape
    return pl.pallas_call(
        functools.partial(_fps_body, S, N),
        grid=(B,),
        in_specs=[pl.BlockSpec((1, 3, N), lambda b: (b, 0, 0))],
        out_specs=pl.BlockSpec((1, S, 3), lambda b: (b, 0, 0)),
        out_shape=jax.ShapeDtypeStruct((B, S, 3), jnp.float32),
    )(xyzT)


def _ballq_body(N, T, K, r2, xyzT_ref, nx_ref, u_ref, tab_ref,
                off_ref, out_ref):
    xr = xyzT_ref[0, 0:1, :]
    yr = xyzT_ref[0, 1:2, :]
    zr = xyzT_ref[0, 2:3, :]
    nx = nx_ref[0]
    d = (nx[:, 0:1] - xr) ** 2
    d = d + (nx[:, 1:2] - yr) ** 2
    d = d + (nx[:, 2:3] - zr) ** 2
    maskT = d <= r2
    cT = jnp.dot(maskT.astype(jnp.bfloat16), u_ref[...],
                 preferred_element_type=jnp.float32)
    kk = (jax.lax.broadcasted_iota(jnp.int32, (K, 1), 0) + 1).astype(
        jnp.float32)
    tab = tab_ref[0]
    hi = tab.astype(jnp.bfloat16)
    mid = (tab - hi.astype(jnp.float32)).astype(jnp.bfloat16)
    for s in range(T):
        c_row = cT[s:s + 1, :]
        m_row = maskT[s:s + 1, :]
        cnt = cT[s:s + 1, N - 1:N]
        oh = m_row & ((c_row == kk) | ((kk > cnt) & (c_row == 1.0)))
        ohb = oh.astype(jnp.bfloat16)
        g = jnp.dot(ohb, hi, preferred_element_type=jnp.float32)
        g = g + jnp.dot(ohb, mid, preferred_element_type=jnp.float32)
        g = g - off_ref[0, s:s + 1, :]
        out_ref[pl.ds(s * K, K), :] = g


def _ballq_gather(xyzT, new_xyz, tab, offsets, radius, K):
    B, _, N = xyzT.shape
    S = new_xyz.shape[1]
    Cp = tab.shape[2]
    T = 8
    u = (jax.lax.broadcasted_iota(jnp.int32, (N, N), 0)
         <= jax.lax.broadcasted_iota(jnp.int32, (N, N), 1)).astype(jnp.bfloat16)
    grid = (B, S // T)
    return pl.pallas_call(
        functools.partial(_ballq_body, N, T, K, radius * radius),
        grid=grid,
        in_specs=[
            pl.BlockSpec((1, 3, N), lambda b, j: (b, 0, 0)),
            pl.BlockSpec((1, T, 3), lambda b, j: (b, j, 0)),
            pl.BlockSpec((N, N), lambda b, j: (0, 0)),
            pl.BlockSpec((1, N, Cp), lambda b, j: (b, 0, 0)),
            pl.BlockSpec((1, T, Cp), lambda b, j: (b, j, 0)),
        ],
        out_specs=pl.BlockSpec((T * K, Cp), lambda b, j: (b * (S // T) + j, 0)),
        out_shape=jax.ShapeDtypeStruct((B * S * K, Cp), jnp.float32),
    )(xyzT, new_xyz, u, tab, offsets)


def _layer_body(M, R, has_bn, x_ref, wt_ref, b_ref, sums_ref, gb_ref,
                y_ref, so_ref):
    x = x_ref[...]
    if has_bn:
        m = sums_ref[0:1, :] * (1.0 / M)
        v = sums_ref[1:2, :] * (1.0 / M) - m * m
        scale = gb_ref[0:1, :] / jnp.sqrt(v + _EPS)
        x = jnp.maximum((x - m) * scale + gb_ref[1:2, :], 0.0)
    y = jnp.dot(x.astype(jnp.bfloat16), wt_ref[...].astype(jnp.bfloat16),
                preferred_element_type=jnp.float32)
    y = y + b_ref[...]
    y_ref[...] = y

    @pl.when(pl.program_id(0) == 0)
    def _():
        so_ref[...] = jnp.zeros_like(so_ref)

    so_ref[0:1, :] += jnp.sum(y, axis=0, keepdims=True)
    so_ref[1:2, :] += jnp.sum(y * y, axis=0, keepdims=True)


def _layer(x, sums, gb_prev, p):
    M, Cin = x.shape
    W, b = p[0], p[1]
    Cout = W.shape[0]
    wt = W.T
    b2 = b.reshape(1, Cout)
    R = M if M <= 2048 else 2048
    assert M % R == 0
    has_bn = gb_prev is not None
    if sums is None:
        sums = jnp.zeros((2, Cin), jnp.float32)
    gb = gb_prev if has_bn else jnp.zeros((2, Cin), jnp.float32)
    return pl.pallas_call(
        functools.partial(_layer_body, M, R, has_bn),
        grid=(M // R,),
        in_specs=[
            pl.BlockSpec((R, Cin), lambda i: (i, 0)),
            pl.BlockSpec((Cin, Cout), lambda i: (0, 0)),
            pl.BlockSpec((1, Cout), lambda i: (0, 0)),
            pl.BlockSpec((2, Cin), lambda i: (0, 0)),
            pl.BlockSpec((2, Cin), lambda i: (0, 0)),
        ],
        out_specs=(
            pl.BlockSpec((R, Cout), lambda i: (i, 0)),
            pl.BlockSpec((2, Cout), lambda i: (0, 0)),
        ),
        out_shape=(
            jax.ShapeDtypeStruct((M, Cout), jnp.float32),
            jax.ShapeDtypeStruct((2, Cout), jnp.float32),
        ),
    )(x, wt, b2, sums, gb)


def _bnmax_body(M, K, Ts, y_ref, sums_ref, gb_ref, out_ref):
    x = y_ref[...]
    m = sums_ref[0:1, :] * (1.0 / M)
    v = sums_ref[1:2, :] * (1.0 / M) - m * m
    scale = gb_ref[0:1, :] / jnp.sqrt(v + _EPS)
    x = jnp.maximum((x - m) * scale + gb_ref[1:2, :], 0.0)
    C = x.shape[1]
    x3 = x.reshape(Ts, K, C)
    out_ref[...] = jnp.max(x3, axis=1)


def _bn_relu_maxpool(y, sums, g, be, K):
    M, C = y.shape
    G = M // K
    Ts = G if G * K <= 2048 else max(2048 // K, 8)
    while G % Ts:
        Ts //= 2
    gb = jnp.stack([g, be])
    return pl.pallas_call(
        functools.partial(_bnmax_body, M, K, Ts),
        grid=(G // Ts,),
        in_specs=[
            pl.BlockSpec((Ts * K, C), lambda i: (i, 0)),
            pl.BlockSpec((2, C), lambda i: (0, 0)),
            pl.BlockSpec((2, C), lambda i: (0, 0)),
        ],
        out_specs=pl.BlockSpec((Ts, C), lambda i: (i, 0)),
        out_shape=jax.ShapeDtypeStruct((G, C), jnp.float32),
    )(y, sums, gb)


def _mlp_branch(x, params, K):
    y, sums = _layer(x, None, None, params[0])
    for i in range(1, len(params)):
        gb_prev = jnp.stack([params[i - 1][2], params[i - 1][3]])
        y, sums = _layer(y, sums, gb_prev, params[i])
    g, be = params[-1][2], params[-1][3]
    return _bn_relu_maxpool(y, sums, g, be, K)


def _split_bf16(t):
    hi = t.astype(jnp.bfloat16)
    mid = (t - hi.astype(jnp.float32)).astype(jnp.bfloat16)
    return hi, mid


def _sa_msg(xyzT, points, npoint, radius_list, nsample_list, params):
    B, _, N = xyzT.shape
    S = npoint
    new_xyz = _fps(xyzT, S)
    xyz = jnp.transpose(xyzT, (0, 2, 1))
    tab = xyz if points is None else jnp.concatenate([points, xyz], axis=-1)
    Cp = tab.shape[2]
    offsets = jnp.concatenate(
        [jnp.zeros((B, S, Cp - 3), jnp.float32), new_xyz], axis=-1)
    outs = []
    for i, (K, radius) in enumerate(zip(nsample_list, radius_list)):
        gp = _ballq_gather(xyzT, new_xyz, tab, offsets, radius, K)
        o = _mlp_branch(gp, params[i], K)
        outs.append(o.reshape(B, S, -1))
    return new_xyz, jnp.concatenate(outs, axis=-1)


def kernel(input_data, sa1_params, sa2_params, sa3_params):
    B = input_data.shape[0]
    xyzT = input_data[:, :3, :]
    norm = jnp.transpose(input_data[:, 3:, :], (0, 2, 1))
    l1_xyz, l1_points = _sa_msg(xyzT, norm, 512, [0.1, 0.2, 0.4],
                                [16, 32, 128], sa1_params)
    l1_xyzT = jnp.transpose(l1_xyz, (0, 2, 1))
    l2_xyz, l2_points = _sa_msg(l1_xyzT, l1_points, 128, [0.2, 0.4, 0.8],
                                [32, 64, 128], sa2_params)
    rows = jnp.concatenate([l2_xyz, l2_points], axis=-1)
    M = B * rows.shape[1]
    x = rows.reshape(M, rows.shape[2])
    out = _mlp_branch(x, sa3_params, rows.shape[1])
    return out.reshape(B, 1024)
